# baseline (speedup 1.0000x reference)
import functools

import jax
import jax.numpy as jnp
from jax import lax
from jax.experimental import pallas as pl
from jax.experimental.pallas import tpu as pltpu
from jax.experimental.pallas import tpu_sc as plsc

VOCAB = 1000000
D = 64
B = 16384

NC = 2
NS = 16
NW = NC * NS
L = 16

B_PER_W = B // NW
CHUNK = 128
NCHUNK = B_PER_W // CHUNK
NGROUP = B_PER_W // L


@functools.partial(
    pl.kernel,
    out_type=jax.ShapeDtypeStruct((B,), jnp.float32),
    mesh=plsc.VectorSubcoreMesh(core_axis_name="c", subcore_axis_name="s"),
    compiler_params=pltpu.CompilerParams(
        needs_layout_passes=False, use_tc_tiling_on_sc=False),
    scratch_types=[
        pltpu.VMEM((NCHUNK, CHUNK), jnp.int32),
        pltpu.VMEM((NCHUNK, CHUNK), jnp.int32),
        pltpu.VMEM((B_PER_W, D), jnp.float32),
        pltpu.VMEM((B_PER_W, D), jnp.float32),
        pltpu.VMEM((B_PER_W,), jnp.float32),
        pltpu.VMEM((B_PER_W,), jnp.float32),
        pltpu.VMEM((B_PER_W,), jnp.float32),
        pltpu.SemaphoreType.DMA,
    ],
)
def _glove_sc(tid_hbm, cid_hbm, w_hbm, c_hbm, wb_hbm, cb_hbm, out_hbm,
              tid_v, cid_v, w_rows, c_rows, wb_v, cb_v, out_v, sem):
    wid = lax.axis_index("s") * NC + lax.axis_index("c")
    base = pl.multiple_of(wid * B_PER_W, B_PER_W)
    crow0 = wid * NCHUNK

    pltpu.sync_copy(tid_hbm.at[pl.ds(crow0, NCHUNK)], tid_v)
    pltpu.sync_copy(cid_hbm.at[pl.ds(crow0, NCHUNK)], cid_v)
    copies = []
    for k in range(NCHUNK):
        o = k * CHUNK
        tidx = tid_v.at[k]
        cidx = cid_v.at[k]
        copies.append(pltpu.async_copy(w_hbm.at[tidx],
                                       w_rows.at[pl.ds(o, CHUNK)], sem))
        copies.append(pltpu.async_copy(c_hbm.at[cidx],
                                       c_rows.at[pl.ds(o, CHUNK)], sem))
        copies.append(pltpu.async_copy(wb_hbm.at[tidx],
                                       wb_v.at[pl.ds(o, CHUNK)], sem))
        copies.append(pltpu.async_copy(cb_hbm.at[cidx],
                                       cb_v.at[pl.ds(o, CHUNK)], sem))
    for cp in copies:
        cp.wait()

    iota16 = lax.iota(jnp.int32, L)

    def body(g, carry):
        go = pl.multiple_of(g * L, L)
        acc = wb_v[pl.ds(go, L)] + cb_v[pl.ds(go, L)]
        rows = go + iota16
        for j in range(D):
            colj = jnp.full((L,), j, jnp.int32)
            acc = acc + (plsc.load_gather(w_rows, [rows, colj])
                         * plsc.load_gather(c_rows, [rows, colj]))
        out_v[pl.ds(go, L)] = acc
        return carry

    lax.fori_loop(0, NGROUP, body, 0)
    pltpu.sync_copy(out_v, out_hbm.at[pl.ds(base, B_PER_W)])


def kernel(target_ids, context_ids, w_emb, c_emb, w_bias, c_bias):
    tid = target_ids.astype(jnp.int32).reshape(NW * NCHUNK, CHUNK)
    cid = context_ids.astype(jnp.int32).reshape(NW * NCHUNK, CHUNK)
    wb = w_bias.reshape(VOCAB)
    cb = c_bias.reshape(VOCAB)
    return _glove_sc(tid, cid, w_emb, c_emb, wb, cb)
